# baseline (speedup 1.0000x reference)
import jax
import jax.numpy as jnp
from jax.experimental import pallas as pl
from jax.experimental.pallas import tpu as pltpu

B, L, DP, DA, N = 4, 512, 128, 128, 1024
K = 32
NB = N // K + B
GRID = 2
SLOTS = NB // GRID
NEG = -1e30


def _rt(a):
    return a.astype(jnp.bfloat16).astype(jnp.float32)


def _attn_body(sched_b, sched_s, sched_lo, sched_hi,
               atom, prot2d, W1, b1, w2row, b2,
               m_out, Yp_out, P1_ref, A1_ref):
    f32 = jnp.float32
    hi_p = jax.lax.Precision.HIGHEST
    W1p = _rt(W1[:DP, :])
    W1a = _rt(W1[DP:, :])
    P1_ref[:, :] = jnp.dot(_rt(prot2d[:, :]), W1p, precision=hi_p,
                           preferred_element_type=f32)
    A1_ref[:, :] = jnp.dot(_rt(atom[:, :]), W1a, precision=hi_p,
                           preferred_element_type=f32) + b1[0:1, :]
    m_out[:, :] = jnp.full((N, 1), NEG, f32)
    Yp_out[:, :] = jnp.full((8, L), NEG, f32)

    w2 = _rt(w2row[0:1, :])
    b2s = b2[0, 0]
    k_iota = jax.lax.broadcasted_iota(jnp.int32, (K, 1), 0)
    base = pl.program_id(0) * SLOTS

    def body(j, _):
        i = base + j
        b = sched_b[i]
        s = sched_s[i]
        lo = sched_lo[i]
        hi = sched_hi[i]
        idx = s + k_iota
        valid = (idx >= lo) & (idx < hi)
        P1b = P1_ref[pl.ds(b * L, L), :]
        A1k = A1_ref[pl.ds(s, K), :]
        T = jnp.tanh(P1b[None, :, :] + A1k[:, None, :])
        y = jnp.sum(_rt(T) * w2[None, :, :], axis=-1) + b2s
        m_k = jnp.max(y, axis=1, keepdims=True)
        old_m = m_out[pl.ds(s, K), :]
        m_out[pl.ds(s, K), :] = jnp.where(valid, m_k, old_m)
        y_mask = jnp.where(valid, y, NEG)
        seg_y = jnp.max(y_mask, axis=0, keepdims=True)
        old_y = Yp_out[pl.ds(b, 1), :]
        Yp_out[pl.ds(b, 1), :] = jnp.maximum(old_y, seg_y)
        return 0

    jax.lax.fori_loop(0, SLOTS, body, 0)


def _final_body(m2, Yp2, atom, prot2d, splits_f,
                Wd1, bd1, Wd2, bd2, Wo, bo, out_ref):
    f32 = jnp.float32
    hi_p = jax.lax.Precision.HIGHEST
    m = jnp.maximum(m2[0:N, :], m2[N:2 * N, :])
    Yp = jnp.maximum(Yp2[0:B, :], Yp2[8:8 + B, :])
    Wc = jnp.exp(5.0 * jnp.tanh(m))
    lane = jax.lax.broadcasted_iota(jnp.int32, (N, 128), 1)
    oh = (lane == splits_f[:, :].astype(jnp.int32)).astype(f32)
    dn = (((0,), (0,)), ((), ()))
    seg_sum = jax.lax.dot_general(oh, Wc, dn, precision=hi_p,
                                  preferred_element_type=f32)
    Sc = jnp.dot(oh, seg_sum, precision=hi_p,
                 preferred_element_type=f32)
    aa = Wc / Sc
    atom_agg = jax.lax.dot_general(oh, aa * atom[:, :], dn, precision=hi_p,
                                   preferred_element_type=f32)
    Wp = 5.0 * jnp.tanh(Yp)
    Wp = Wp - jnp.max(Wp, axis=1, keepdims=True)
    e = jnp.exp(Wp)
    ap = e / jnp.sum(e, axis=1, keepdims=True)
    pe = []
    for bb in range(B):
        pe.append(jnp.dot(_rt(ap[bb:bb + 1, :]),
                          _rt(prot2d[bb * L:(bb + 1) * L, :]), precision=hi_p,
                          preferred_element_type=f32))
    prot_embed = jnp.concatenate(pe, axis=0)
    x = jnp.concatenate([atom_agg[:B, :], prot_embed], axis=1)
    x = jnp.maximum(jnp.dot(_rt(x), _rt(Wd1[:, :]), precision=hi_p,
                            preferred_element_type=f32)
                    + bd1[0:1, :], 0.0)
    x = jnp.maximum(jnp.dot(_rt(x), _rt(Wd2[:, :]), precision=hi_p,
                            preferred_element_type=f32)
                    + bd2[0:1, :], 0.0)
    res = jnp.dot(_rt(x), _rt(Wo[:, :]), precision=hi_p,
                  preferred_element_type=f32) + bo[0, 0]
    out_ref[0:B, 0:1] = res


def _build_schedule(atom_splits):
    i32 = jnp.int32
    counts = jnp.bincount(atom_splits, length=B).astype(i32)
    ends = jnp.cumsum(counts)
    starts = ends - counts
    blocks_per = (counts + K - 1) // K
    cumb = jnp.cumsum(blocks_per)
    i = jnp.arange(NB, dtype=i32)
    b_of = jnp.searchsorted(cumb, i, side="right").astype(i32)
    real = b_of < B
    b_idx = jnp.minimum(b_of, B - 1)
    prev = jnp.where(b_idx > 0, cumb[jnp.maximum(b_idx - 1, 0)], 0)
    j = i - prev
    s = jnp.minimum(starts[b_idx] + j * K, N - K)
    sched_b = jnp.where(real, b_idx, 0).astype(i32)
    sched_s = jnp.where(real, s, 0).astype(i32)
    sched_lo = jnp.where(real, starts[b_idx], 0).astype(i32)
    sched_hi = jnp.where(real, ends[b_idx], 0).astype(i32)
    return sched_b, sched_s, sched_lo, sched_hi


@jax.jit
def kernel(atom_embed, protSeq_embed, atom_splits, W1, b1, W2, b2,
           Wd1, bd1, Wd2, bd2, Wo, bo):
    f32 = jnp.float32
    sched_b, sched_s, sched_lo, sched_hi = _build_schedule(atom_splits)
    prot2d = protSeq_embed.reshape(B * L, DP)
    splits_f = atom_splits.reshape(N, 1)
    smem = pl.BlockSpec(memory_space=pltpu.SMEM)
    vmem = pl.BlockSpec(memory_space=pltpu.VMEM)
    m2, Yp2 = pl.pallas_call(
        _attn_body,
        grid=(GRID,),
        out_shape=(jax.ShapeDtypeStruct((GRID * N, 1), f32),
                   jax.ShapeDtypeStruct((GRID * 8, L), f32)),
        in_specs=[smem, smem, smem, smem] + [vmem] * 6,
        out_specs=(pl.BlockSpec((N, 1), lambda i: (i, 0)),
                   pl.BlockSpec((8, L), lambda i: (i, 0))),
        scratch_shapes=[
            pltpu.VMEM((B * L, 128), f32),
            pltpu.VMEM((N, 128), f32),
        ],
        compiler_params=pltpu.CompilerParams(
            dimension_semantics=("parallel",)),
    )(sched_b, sched_s, sched_lo, sched_hi,
      atom_embed, prot2d,
      W1, b1.reshape(1, 128), W2.reshape(1, 128), b2.reshape(1, 1))
    out = pl.pallas_call(
        _final_body,
        out_shape=jax.ShapeDtypeStruct((8, 128), f32),
        in_specs=[vmem] * 11,
        out_specs=vmem,
    )(m2, Yp2, atom_embed, prot2d, splits_f,
      Wd1, bd1.reshape(1, Wd1.shape[1]), Wd2,
      bd2.reshape(1, Wd2.shape[1]), Wo, bo.reshape(1, 1))
    return out[:B, :1]


# speedup vs baseline: 1.0800x; 1.0800x over previous
import functools

import jax
import jax.numpy as jnp
from jax.experimental import pallas as pl
from jax.experimental.pallas import tpu as pltpu

B, L, DP, DA, N = 4, 512, 128, 128, 1024
K = 32
NB = N // K + B
NEG = -1e30


def _kernel_body(nblk, sched_b, sched_s, sched_lo, sched_hi,
                 atom, prot2d, splits_f, W1, b1, w2row, b2,
                 Wd1, bd1, Wd2, bd2, Wo, bo,
                 out_ref, P1_ref, A1_ref, m_ref, Yp_ref):
    f32 = jnp.float32
    rt = lambda a: a.astype(jnp.bfloat16).astype(f32)
    hi = jax.lax.Precision.HIGHEST
    W1p = rt(W1[:DP, :])
    W1a = rt(W1[DP:, :])
    P1_ref[:, :] = jnp.dot(rt(prot2d[:, :]), W1p, precision=hi,
                           preferred_element_type=f32)
    A1_ref[:, :] = jnp.dot(rt(atom[:, :]), W1a, precision=hi,
                           preferred_element_type=f32) + b1[0:1, :]
    Yp_ref[:, :] = jnp.full((B, L), NEG, f32)

    w2 = rt(w2row[0:1, :])
    b2s = b2[0, 0]
    k_iota = jax.lax.broadcasted_iota(jnp.int32, (K, 1), 0)

    def body(i, _):
        b = sched_b[i]
        s = sched_s[i]
        lo = sched_lo[i]
        hi = sched_hi[i]
        idx = s + k_iota
        valid = (idx >= lo) & (idx < hi)
        P1b = P1_ref[pl.ds(b * L, L), :]
        A1k = A1_ref[pl.ds(s, K), :]
        T = jnp.tanh(P1b[None, :, :] + A1k[:, None, :])
        y = jnp.sum(rt(T) * w2[None, :, :], axis=-1) + b2s
        m_k = jnp.max(y, axis=1, keepdims=True)
        old_m = m_ref[pl.ds(s, K), :]
        m_ref[pl.ds(s, K), :] = jnp.where(valid, m_k, old_m)
        y_mask = jnp.where(valid, y, NEG)
        seg_y = jnp.max(y_mask, axis=0, keepdims=True)
        old_y = Yp_ref[pl.ds(b, 1), :]
        Yp_ref[pl.ds(b, 1), :] = jnp.maximum(old_y, seg_y)
        return 0

    jax.lax.fori_loop(0, nblk[0], body, 0)

    m = m_ref[:, :]
    Wc = jnp.exp(5.0 * jnp.tanh(m))
    lane = jax.lax.broadcasted_iota(jnp.int32, (N, 128), 1)
    oh = (lane == splits_f[:, :].astype(jnp.int32)).astype(f32)
    dn = (((0,), (0,)), ((), ()))
    seg_sum = jax.lax.dot_general(oh, Wc, dn, precision=hi,
                                  preferred_element_type=f32)
    Sc = jnp.dot(oh, seg_sum, precision=hi,
                 preferred_element_type=f32)
    aa = Wc / Sc
    atom_agg = jax.lax.dot_general(oh, aa * atom[:, :], dn, precision=hi,
                                   preferred_element_type=f32)
    Wp = 5.0 * jnp.tanh(Yp_ref[:, :])
    Wp = Wp - jnp.max(Wp, axis=1, keepdims=True)
    e = jnp.exp(Wp)
    ap = e / jnp.sum(e, axis=1, keepdims=True)
    pe = []
    for bb in range(B):
        pe.append(jnp.dot(rt(ap[bb:bb + 1, :]),
                          rt(prot2d[bb * L:(bb + 1) * L, :]), precision=hi,
                          preferred_element_type=f32))
    prot_embed = jnp.concatenate(pe, axis=0)
    x = jnp.concatenate([atom_agg[:B, :], prot_embed], axis=1)
    x = jnp.maximum(jnp.dot(rt(x), rt(Wd1[:, :]), precision=hi,
                            preferred_element_type=f32)
                    + bd1[0:1, :], 0.0)
    x = jnp.maximum(jnp.dot(rt(x), rt(Wd2[:, :]), precision=hi,
                            preferred_element_type=f32)
                    + bd2[0:1, :], 0.0)
    res = jnp.dot(rt(x), rt(Wo[:, :]), precision=hi,
                  preferred_element_type=f32) + bo[0, 0]
    out_ref[0:B, 0:1] = res


def _build_schedule(atom_splits):
    i32 = jnp.int32
    counts = jnp.bincount(atom_splits, length=B).astype(i32)
    ends = jnp.cumsum(counts)
    starts = ends - counts
    blocks_per = (counts + K - 1) // K
    cumb = jnp.cumsum(blocks_per)
    i = jnp.arange(NB, dtype=i32)
    b_of = jnp.searchsorted(cumb, i, side="right").astype(i32)
    real = b_of < B
    b_idx = jnp.minimum(b_of, B - 1)
    prev = jnp.where(b_idx > 0, cumb[jnp.maximum(b_idx - 1, 0)], 0)
    j = i - prev
    s = jnp.minimum(starts[b_idx] + j * K, N - K)
    sched_b = jnp.where(real, b_idx, 0).astype(i32)
    sched_s = jnp.where(real, s, 0).astype(i32)
    sched_lo = jnp.where(real, starts[b_idx], 0).astype(i32)
    sched_hi = jnp.where(real, ends[b_idx], 0).astype(i32)
    nblk = cumb[B - 1:B].astype(i32)
    return nblk, sched_b, sched_s, sched_lo, sched_hi


@jax.jit
def kernel(atom_embed, protSeq_embed, atom_splits, W1, b1, W2, b2,
           Wd1, bd1, Wd2, bd2, Wo, bo):
    f32 = jnp.float32
    nblk, sched_b, sched_s, sched_lo, sched_hi = _build_schedule(atom_splits)
    prot2d = protSeq_embed.reshape(B * L, DP)
    splits_f = atom_splits.reshape(N, 1)
    smem = pl.BlockSpec(memory_space=pltpu.SMEM)
    vmem = pl.BlockSpec(memory_space=pltpu.VMEM)
    out = pl.pallas_call(
        _kernel_body,
        out_shape=jax.ShapeDtypeStruct((8, 128), f32),
        in_specs=[smem, smem, smem, smem, smem] + [vmem] * 13,
        out_specs=vmem,
        scratch_shapes=[
            pltpu.VMEM((B * L, 128), f32),
            pltpu.VMEM((N, 128), f32),
            pltpu.VMEM((N, 1), f32),
            pltpu.VMEM((B, L), f32),
        ],
    )(nblk, sched_b, sched_s, sched_lo, sched_hi,
      atom_embed, prot2d, splits_f,
      W1, b1.reshape(1, 128), W2.reshape(1, 128),
      b2.reshape(1, 1),
      Wd1, bd1.reshape(1, Wd1.shape[1]), Wd2,
      bd2.reshape(1, Wd2.shape[1]), Wo, bo.reshape(1, 1))
    return out[:B, :1]


# speedup vs baseline: 1.0821x; 1.0020x over previous
import functools

import jax
import jax.numpy as jnp
from jax.experimental import pallas as pl
from jax.experimental.pallas import tpu as pltpu

B, L, DP, DA, N = 4, 512, 128, 128, 1024
K = 32
NB = N // K + B
NEG = -1e30


def _kernel_body(nblk, sched_b, sched_s, sched_lo, sched_hi,
                 atom, prot2d, splits_f, W1, b1, w2row, b2,
                 Wd1, bd1, Wd2, bd2, Wo, bo,
                 out_ref, P1_ref, A1_ref, m_ref, Yp_ref):
    f32 = jnp.float32
    rt = lambda a: a.astype(jnp.bfloat16).astype(f32)
    hi = jax.lax.Precision.HIGHEST
    W1p = rt(W1[:DP, :])
    W1a = rt(W1[DP:, :])
    P1_ref[:, :] = jnp.dot(rt(prot2d[:, :]), W1p, precision=hi,
                           preferred_element_type=f32)
    A1_ref[:, :] = jnp.dot(rt(atom[:, :]), W1a, precision=hi,
                           preferred_element_type=f32) + b1[0:1, :]
    Yp_ref[:, :] = jnp.full((B, L), NEG, f32)

    w2 = rt(w2row[0:1, :])
    b2s = b2[0, 0]
    k_iota = jax.lax.broadcasted_iota(jnp.int32, (K, 1), 0)

    def body(i, _):
        b = sched_b[i]
        s = sched_s[i]
        lo = sched_lo[i]
        hi = sched_hi[i]
        idx = s + k_iota
        valid = (idx >= lo) & (idx < hi)
        P1b = P1_ref[pl.ds(b * L, L), :]
        A1k = A1_ref[pl.ds(s, K), :]
        T = jnp.tanh(P1b[None, :, :] + A1k[:, None, :])
        y = jnp.sum(T * w2[None, :, :], axis=-1) + b2s
        m_k = jnp.max(y, axis=1, keepdims=True)
        old_m = m_ref[pl.ds(s, K), :]
        m_ref[pl.ds(s, K), :] = jnp.where(valid, m_k, old_m)
        y_mask = jnp.where(valid, y, NEG)
        seg_y = jnp.max(y_mask, axis=0, keepdims=True)
        old_y = Yp_ref[pl.ds(b, 1), :]
        Yp_ref[pl.ds(b, 1), :] = jnp.maximum(old_y, seg_y)
        return 0

    jax.lax.fori_loop(0, nblk[0], body, 0)

    m = m_ref[:, :]
    Wc = jnp.exp(5.0 * jnp.tanh(m))
    lane = jax.lax.broadcasted_iota(jnp.int32, (N, 128), 1)
    oh = (lane == splits_f[:, :].astype(jnp.int32)).astype(f32)
    dn = (((0,), (0,)), ((), ()))
    seg_sum = jax.lax.dot_general(oh, Wc, dn, precision=hi,
                                  preferred_element_type=f32)
    Sc = jnp.dot(oh, seg_sum, precision=hi,
                 preferred_element_type=f32)
    aa = Wc / Sc
    atom_agg = jax.lax.dot_general(oh, aa * atom[:, :], dn, precision=hi,
                                   preferred_element_type=f32)
    Wp = 5.0 * jnp.tanh(Yp_ref[:, :])
    Wp = Wp - jnp.max(Wp, axis=1, keepdims=True)
    e = jnp.exp(Wp)
    ap = e / jnp.sum(e, axis=1, keepdims=True)
    pe = []
    for bb in range(B):
        pe.append(jnp.dot(rt(ap[bb:bb + 1, :]),
                          rt(prot2d[bb * L:(bb + 1) * L, :]), precision=hi,
                          preferred_element_type=f32))
    prot_embed = jnp.concatenate(pe, axis=0)
    x = jnp.concatenate([atom_agg[:B, :], prot_embed], axis=1)
    x = jnp.maximum(jnp.dot(rt(x), rt(Wd1[:, :]), precision=hi,
                            preferred_element_type=f32)
                    + bd1[0:1, :], 0.0)
    x = jnp.maximum(jnp.dot(rt(x), rt(Wd2[:, :]), precision=hi,
                            preferred_element_type=f32)
                    + bd2[0:1, :], 0.0)
    res = jnp.dot(rt(x), rt(Wo[:, :]), precision=hi,
                  preferred_element_type=f32) + bo[0, 0]
    out_ref[0:B, 0:1] = res


def _build_schedule(atom_splits):
    i32 = jnp.int32
    counts = jnp.bincount(atom_splits, length=B).astype(i32)
    ends = jnp.cumsum(counts)
    starts = ends - counts
    blocks_per = (counts + K - 1) // K
    cumb = jnp.cumsum(blocks_per)
    i = jnp.arange(NB, dtype=i32)
    b_of = jnp.searchsorted(cumb, i, side="right").astype(i32)
    real = b_of < B
    b_idx = jnp.minimum(b_of, B - 1)
    prev = jnp.where(b_idx > 0, cumb[jnp.maximum(b_idx - 1, 0)], 0)
    j = i - prev
    s = jnp.minimum(starts[b_idx] + j * K, N - K)
    sched_b = jnp.where(real, b_idx, 0).astype(i32)
    sched_s = jnp.where(real, s, 0).astype(i32)
    sched_lo = jnp.where(real, starts[b_idx], 0).astype(i32)
    sched_hi = jnp.where(real, ends[b_idx], 0).astype(i32)
    nblk = cumb[B - 1:B].astype(i32)
    return nblk, sched_b, sched_s, sched_lo, sched_hi


@jax.jit
def kernel(atom_embed, protSeq_embed, atom_splits, W1, b1, W2, b2,
           Wd1, bd1, Wd2, bd2, Wo, bo):
    f32 = jnp.float32
    nblk, sched_b, sched_s, sched_lo, sched_hi = _build_schedule(atom_splits)
    prot2d = protSeq_embed.reshape(B * L, DP)
    splits_f = atom_splits.reshape(N, 1)
    smem = pl.BlockSpec(memory_space=pltpu.SMEM)
    vmem = pl.BlockSpec(memory_space=pltpu.VMEM)
    out = pl.pallas_call(
        _kernel_body,
        out_shape=jax.ShapeDtypeStruct((8, 128), f32),
        in_specs=[smem, smem, smem, smem, smem] + [vmem] * 13,
        out_specs=vmem,
        scratch_shapes=[
            pltpu.VMEM((B * L, 128), f32),
            pltpu.VMEM((N, 128), f32),
            pltpu.VMEM((N, 1), f32),
            pltpu.VMEM((B, L), f32),
        ],
    )(nblk, sched_b, sched_s, sched_lo, sched_hi,
      atom_embed, prot2d, splits_f,
      W1, b1.reshape(1, 128), W2.reshape(1, 128),
      b2.reshape(1, 1),
      Wd1, bd1.reshape(1, Wd1.shape[1]), Wd2,
      bd2.reshape(1, Wd2.shape[1]), Wo, bo.reshape(1, 1))
    return out[:B, :1]
